# baseline (speedup 1.0000x reference)
import jax
import jax.numpy as jnp
from jax import lax
from jax.experimental import pallas as pl
from jax.experimental.pallas import tpu as pltpu
from jax.experimental.pallas import tpu_sc as plsc

_N = 50000
_E = 800000
_G = 128
_D = 64
_HALF = _N // 2
_NDUMP = 88
_RD = _HALF + _NDUMP
_STRIPE = _RD // 16
_K = 128
_GRP = 2
_MAC = 6
_NBUF = 3
_ET = _E + _N
_ROWS = 6720
_ET_PAD = _ROWS * _K
_RPT = _ROWS // 16
_OUTER = _RPT // _GRP

_BM = 2000
_TGRID = _N // _BM

_mesh = plsc.VectorSubcoreMesh(
    core_axis_name="c", subcore_axis_name="s", num_cores=2, num_subcores=16
)


def _deg_body(dst_hbm, ones_hbm, zcol_hbm, deg_hbm, dstbuf, idxbuf, onesv, dacc):
    c = lax.axis_index("c")
    s = lax.axis_index("s")
    pltpu.sync_copy(zcol_hbm, dacc.at[pl.ds(s * _STRIPE, _STRIPE)])
    pltpu.sync_copy(ones_hbm, onesv)
    plsc.subcore_barrier()
    row_base = s * _RPT

    def chunk(j, carry):
        r0 = row_base + j * _GRP
        pltpu.sync_copy(dst_hbm.at[pl.ds(r0, _GRP)], dstbuf)
        for g in range(_GRP):
            for i in range(8):
                dv = dstbuf[g, pl.ds(i * 16, 16)]
                loc = dv - c * _HALF
                inb = (loc >= 0) & (loc < _HALF)
                dmp = _HALF + lax.iota(jnp.int32, 16) + ((g + i) % 4) * 16
                idxbuf[g, pl.ds(i * 16, 16)] = jnp.where(inb, loc, dmp)
        for g in range(_GRP):
            pltpu.sync_copy(onesv, dacc.at[idxbuf.at[g]], add=True)
        return carry

    lax.fori_loop(0, _OUTER, chunk, 0)
    plsc.subcore_barrier()

    @pl.when(s < 15)
    def _():
        pltpu.sync_copy(
            dacc.at[pl.ds(s * _STRIPE, _STRIPE)],
            deg_hbm.at[pl.ds(c * _HALF + s * _STRIPE, _STRIPE)],
        )

    @pl.when(s == 15)
    def _():
        pltpu.sync_copy(
            dacc.at[pl.ds(15 * _STRIPE, _HALF - 15 * _STRIPE)],
            deg_hbm.at[pl.ds(c * _HALF + 15 * _STRIPE, _HALF - 15 * _STRIPE)],
        )


def _agg_body(h_hbm, src_hbm, dst_hbm, zrows_hbm, out_hbm,
              srcv, dstraw, dstv, rows, acc,
              isem0, isem1, gsem0, gsem1, gsem2, ssem0, ssem1, ssem2):
    c = lax.axis_index("c")
    s = lax.axis_index("s")
    isem = (isem0, isem1)
    gsem = (gsem0, gsem1, gsem2)
    ssem = (ssem0, ssem1, ssem2)
    pltpu.sync_copy(zrows_hbm, acc.at[pl.ds(s * _STRIPE, _STRIPE)])
    plsc.subcore_barrier()
    rb = s * _RPT
    nmac = _RPT // _MAC

    def idx_start(q, a):
        pltpu.async_copy(src_hbm.at[pl.ds(rb + a * _MAC, _MAC)],
                         srcv.at[q], isem[q])
        pltpu.async_copy(dst_hbm.at[pl.ds(rb + a * _MAC, _MAC)],
                         dstraw.at[q], isem[q])

    def idx_wait(q, a):
        pltpu.make_async_copy(src_hbm.at[pl.ds(rb + a * _MAC, _MAC)],
                              srcv.at[q], isem[q]).wait()
        pltpu.make_async_copy(dst_hbm.at[pl.ds(rb + a * _MAC, _MAC)],
                              dstraw.at[q], isem[q]).wait()

    def gather_start(q, i):
        pltpu.async_copy(h_hbm.at[srcv.at[q, i]], rows.at[i % _NBUF],
                         gsem[i % _NBUF])

    def gather_wait(q, i):
        pltpu.make_async_copy(h_hbm.at[srcv.at[q, i]], rows.at[i % _NBUF],
                              gsem[i % _NBUF]).wait()

    def scat_start(q, i):
        pltpu.async_copy(rows.at[i % _NBUF], acc.at[dstv.at[q, i]],
                         ssem[i % _NBUF], add=True)

    def scat_wait(q, i):
        pltpu.make_async_copy(rows.at[i % _NBUF], acc.at[dstv.at[q, i]],
                              ssem[i % _NBUF]).wait()

    idx_start(0, 0)

    def macro(m, q):
        a = m + q

        idx_wait(q, a)
        for i in range(_MAC):
            for v in range(8):
                dv = dstraw[q, i, pl.ds(v * 16, 16)]
                loc = dv - c * _HALF
                inb = (loc >= 0) & (loc < _HALF)
                dmp = _HALF + lax.iota(jnp.int32, 16) + ((i + v) % 4) * 16
                dstv[q, i, pl.ds(v * 16, 16)] = jnp.where(inb, loc, dmp)

        for i in range(_MAC):
            if i >= _NBUF:
                scat_wait(q, i - _NBUF)
            else:
                @pl.when(a >= 1)
                def _():
                    scat_wait(1 - q, _MAC - _NBUF + i)
            gather_start(q, i)
            if i >= 2:
                gather_wait(q, i - 2)
                scat_start(q, i - 2)
            else:
                @pl.when(a >= 1)
                def _():
                    gather_wait(1 - q, _MAC - 2 + i)
                    scat_start(1 - q, _MAC - 2 + i)
            if i == 1:
                @pl.when(a + 1 < nmac)
                def _():
                    idx_start(1 - q, a + 1)

        return 0

    def macro_pair(m, carry):
        macro(m, 0)
        macro(m, 1)
        return carry

    lax.fori_loop(0, nmac // 2, lambda t, cc: macro_pair(t * 2, cc), 0)
    qlast = (nmac - 1) % 2
    for i in (_MAC - 2, _MAC - 1):
        gather_wait(qlast, i)
        scat_start(qlast, i)
    for i in (_MAC - 3, _MAC - 2, _MAC - 1):
        scat_wait(qlast, i)
    plsc.subcore_barrier()

    @pl.when(s < 15)
    def _():
        pltpu.sync_copy(
            acc.at[pl.ds(s * _STRIPE, _STRIPE)],
            out_hbm.at[pl.ds(c * _HALF + s * _STRIPE, _STRIPE)],
        )

    @pl.when(s == 15)
    def _():
        pltpu.sync_copy(
            acc.at[pl.ds(15 * _STRIPE, _HALF - 15 * _STRIPE)],
            out_hbm.at[pl.ds(c * _HALF + 15 * _STRIPE, _HALF - 15 * _STRIPE)],
        )


_sc_params = pltpu.CompilerParams(use_tc_tiling_on_sc=False)


def _make_deg_call():
    return pl.kernel(
        _deg_body,
        out_type=jax.ShapeDtypeStruct((_N, 1), jnp.float32),
        mesh=_mesh,
        compiler_params=_sc_params,
        scratch_types=[
            pltpu.VMEM((_GRP, _K), jnp.int32),
            pltpu.VMEM((_GRP, _K), jnp.int32),
            pltpu.VMEM((_K, 1), jnp.float32),
            pltpu.VMEM_SHARED((_RD, 1), jnp.float32),
        ],
    )


def _make_agg_call():
    return pl.kernel(
        _agg_body,
        out_type=jax.ShapeDtypeStruct((_N, _D), jnp.float32),
        mesh=_mesh,
        compiler_params=_sc_params,
        scratch_types=[
            pltpu.VMEM((2, _MAC, _K), jnp.int32),
            pltpu.VMEM((2, _MAC, _K), jnp.int32),
            pltpu.VMEM((2, _MAC, _K), jnp.int32),
            pltpu.VMEM((_NBUF, _K, _D), jnp.float32),
            pltpu.VMEM_SHARED((_RD, _D), jnp.float32),
        ] + [pltpu.SemaphoreType.DMA] * 8,
    )


def _prologue_body(x_ref, w_ref, deg_ref, h_ref, dinv_ref):
    di = lax.rsqrt(deg_ref[...])
    dinv_ref[...] = di
    h = jnp.dot(x_ref[...], w_ref[...], preferred_element_type=jnp.float32)
    h_ref[...] = h * di


def _stats_body(a_ref, dinv_ref, b_ref, st_ref):
    y = jnp.maximum(a_ref[...] * dinv_ref[...] + b_ref[...], 0.0)
    s1 = jnp.sum(y, axis=0, keepdims=True)
    s2 = jnp.sum(y * y, axis=0, keepdims=True)
    blk = jnp.concatenate(
        [s1, s2, jnp.zeros((6, _D), jnp.float32)], axis=0
    )

    @pl.when(pl.program_id(0) == 0)
    def _():
        st_ref[...] = blk

    @pl.when(pl.program_id(0) > 0)
    def _():
        st_ref[...] = st_ref[...] + blk


def _apply_body(a_ref, dinv_ref, b_ref, st_ref, g_ref, be_ref, w_ref, o_ref):
    di = dinv_ref[...]
    y = jnp.maximum(a_ref[...] * di + b_ref[...], 0.0)
    mean = st_ref[0:1, :] * (1.0 / _N)
    var = st_ref[1:2, :] * (1.0 / _N) - mean * mean
    aa = g_ref[...] * lax.rsqrt(var + 1e-5)
    cc = be_ref[...] - mean * aa
    z = y * aa + cc
    o_ref[...] = jnp.dot(z, w_ref[...], preferred_element_type=jnp.float32) * di


def _pool_body(a_ref, dinv_ref, b_ref, st_ref, g_ref, be_ref, bt_ref, o_ref):
    y = jnp.maximum(a_ref[...] * dinv_ref[...] + b_ref[...], 0.0)
    mean = st_ref[0:1, :] * (1.0 / _N)
    var = st_ref[1:2, :] * (1.0 / _N) - mean * mean
    aa = g_ref[...] * lax.rsqrt(var + 1e-5)
    cc = be_ref[...] - mean * aa
    z = y * aa + cc
    onehot = (bt_ref[...] == lax.broadcasted_iota(jnp.int32, (1, _G), 1))
    part = lax.dot_general(
        onehot.astype(jnp.float32), z, (((0,), (0,)), ((), ())),
        preferred_element_type=jnp.float32,
    )

    @pl.when(pl.program_id(0) == 0)
    def _():
        o_ref[...] = part

    @pl.when(pl.program_id(0) > 0)
    def _():
        o_ref[...] = o_ref[...] + part


def _row_spec(width):
    return pl.BlockSpec((_BM, width), lambda i: (i, 0))


def _full_spec(shape):
    return pl.BlockSpec(shape, lambda i: tuple(0 for _ in shape))


def _prologue_call(x, w1, deg):
    nf = x.shape[1]
    return pl.pallas_call(
        _prologue_body,
        grid=(_TGRID,),
        in_specs=[_row_spec(nf), _full_spec((nf, _D)), _row_spec(1)],
        out_specs=[_row_spec(_D), _row_spec(1)],
        out_shape=[
            jax.ShapeDtypeStruct((_N, _D), jnp.float32),
            jax.ShapeDtypeStruct((_N, 1), jnp.float32),
        ],
    )(x, w1, deg)


def _stats_call(acc, dinv, b):
    return pl.pallas_call(
        _stats_body,
        grid=(_TGRID,),
        in_specs=[_row_spec(_D), _row_spec(1), _full_spec((1, _D))],
        out_specs=_full_spec((8, _D)),
        out_shape=jax.ShapeDtypeStruct((8, _D), jnp.float32),
    )(acc, dinv, b)


def _apply_call(acc, dinv, b, st, g, be, w):
    return pl.pallas_call(
        _apply_body,
        grid=(_TGRID,),
        in_specs=[
            _row_spec(_D), _row_spec(1), _full_spec((1, _D)),
            _full_spec((8, _D)), _full_spec((1, _D)), _full_spec((1, _D)),
            _full_spec((_D, _D)),
        ],
        out_specs=_row_spec(_D),
        out_shape=jax.ShapeDtypeStruct((_N, _D), jnp.float32),
    )(acc, dinv, b, st, g, be, w)


def _pool_call(acc, dinv, b, st, g, be, batch2d):
    return pl.pallas_call(
        _pool_body,
        grid=(_TGRID,),
        in_specs=[
            _row_spec(_D), _row_spec(1), _full_spec((1, _D)),
            _full_spec((8, _D)), _full_spec((1, _D)), _full_spec((1, _D)),
            _row_spec(1),
        ],
        out_specs=_full_spec((_G, _D)),
        out_shape=jax.ShapeDtypeStruct((_G, _D), jnp.float32),
    )(acc, dinv, b, st, g, be, batch2d)


def kernel(x, edge_index, batch, W1, b1, g1, be1, W2, b2, g2, be2,
           W3, b3, g3, be3, W4, b4, g4, be4):
    loop = jnp.arange(_N, dtype=jnp.int32)
    src = jnp.concatenate([edge_index[0].astype(jnp.int32), loop])
    dst = jnp.concatenate([edge_index[1].astype(jnp.int32), loop])
    pad = _ET_PAD - _ET
    src2d = jnp.concatenate([src, jnp.zeros((pad,), jnp.int32)]).reshape(_ROWS, _K)
    dst2d = jnp.concatenate([dst, jnp.full((pad,), _N, jnp.int32)]).reshape(_ROWS, _K)

    ones_col = jnp.ones((_K, 1), jnp.float32)
    zcol = jnp.zeros((_STRIPE, 1), jnp.float32)
    zrows = jnp.zeros((_STRIPE, _D), jnp.float32)
    batch2d = batch.astype(jnp.int32).reshape(_N, 1)

    deg = _make_deg_call()(dst2d, ones_col, zcol)
    h, dinv = _prologue_call(x, W1, deg)

    agg = _make_agg_call()
    params = [(b1, g1, be1, W2), (b2, g2, be2, W3), (b3, g3, be3, W4)]
    for b, g, be, w_next in params:
        a = agg(h, src2d, dst2d, zrows)
        st = _stats_call(a, dinv, b.reshape(1, _D))
        h = _apply_call(a, dinv, b.reshape(1, _D), st, g.reshape(1, _D),
                        be.reshape(1, _D), w_next)
    a = agg(h, src2d, dst2d, zrows)
    st = _stats_call(a, dinv, b4.reshape(1, _D))
    return _pool_call(a, dinv, b4.reshape(1, _D), st, g4.reshape(1, _D),
                      be4.reshape(1, _D), batch2d)


# speedup vs baseline: 1.4113x; 1.4113x over previous
import jax
import jax.numpy as jnp
from jax import lax
from jax.experimental import pallas as pl
from jax.experimental.pallas import tpu as pltpu
from jax.experimental.pallas import tpu_sc as plsc

_N = 50000
_E = 800000
_G = 128
_D = 64
_HALF = _N // 2
_NDUMP = 88
_RD = _HALF + _NDUMP
_STRIPE = _RD // 16
_K = 128
_GRP = 2
_MAC = 4
_NBUF = 2
_ET = _E + _N
_ROWS = 6656
_ET_PAD = _ROWS * _K
_RPT = _ROWS // 16
_OUTER = _RPT // _GRP

_BM = 2000
_TGRID = _N // _BM

_mesh = plsc.VectorSubcoreMesh(
    core_axis_name="c", subcore_axis_name="s", num_cores=2, num_subcores=16
)


def _deg_body(dst_hbm, ones_hbm, zcol_hbm, deg_hbm, dstbuf, idxbuf, onesv, dacc):
    c = lax.axis_index("c")
    s = lax.axis_index("s")
    pltpu.sync_copy(zcol_hbm, dacc.at[pl.ds(s * _STRIPE, _STRIPE)])
    pltpu.sync_copy(ones_hbm, onesv)
    plsc.subcore_barrier()
    row_base = s * _RPT

    def chunk(j, carry):
        r0 = row_base + j * _GRP
        pltpu.sync_copy(dst_hbm.at[pl.ds(r0, _GRP)], dstbuf)
        for g in range(_GRP):
            for i in range(8):
                dv = dstbuf[g, pl.ds(i * 16, 16)]
                loc = dv - c * _HALF
                inb = (loc >= 0) & (loc < _HALF)
                dmp = _HALF + lax.iota(jnp.int32, 16) + ((g + i) % 4) * 16
                idxbuf[g, pl.ds(i * 16, 16)] = jnp.where(inb, loc, dmp)
        for g in range(_GRP):
            pltpu.sync_copy(onesv, dacc.at[idxbuf.at[g]], add=True)
        return carry

    lax.fori_loop(0, _OUTER, chunk, 0)
    plsc.subcore_barrier()

    @pl.when(s < 15)
    def _():
        pltpu.sync_copy(
            dacc.at[pl.ds(s * _STRIPE, _STRIPE)],
            deg_hbm.at[pl.ds(c * _HALF + s * _STRIPE, _STRIPE)],
        )

    @pl.when(s == 15)
    def _():
        pltpu.sync_copy(
            dacc.at[pl.ds(15 * _STRIPE, _HALF - 15 * _STRIPE)],
            deg_hbm.at[pl.ds(c * _HALF + 15 * _STRIPE, _HALF - 15 * _STRIPE)],
        )


def _agg_body(h_hbm, src_hbm, dst_hbm, zrows_hbm, out_hbm,
              srcv, dstraw, dstv, rows, acc,
              isem0, isem1, gsem0, gsem1, gsem2, ssem0, ssem1, ssem2):
    c = lax.axis_index("c")
    s = lax.axis_index("s")
    isem = (isem0, isem1)
    gsem = (gsem0, gsem1, gsem2)
    ssem = (ssem0, ssem1, ssem2)
    pltpu.sync_copy(zrows_hbm, acc.at[pl.ds(s * _STRIPE, _STRIPE)])
    plsc.subcore_barrier()
    rb = s * _RPT
    nmac = _RPT // _MAC

    def idx_start(q, a):
        pltpu.async_copy(src_hbm.at[pl.ds(rb + a * _MAC, _MAC)],
                         srcv.at[q], isem[q])
        pltpu.async_copy(dst_hbm.at[pl.ds(rb + a * _MAC, _MAC)],
                         dstraw.at[q], isem[q])

    def idx_wait(q, a):
        pltpu.make_async_copy(src_hbm.at[pl.ds(rb + a * _MAC, _MAC)],
                              srcv.at[q], isem[q]).wait()
        pltpu.make_async_copy(dst_hbm.at[pl.ds(rb + a * _MAC, _MAC)],
                              dstraw.at[q], isem[q]).wait()

    def gather_start(q, i):
        pltpu.async_copy(h_hbm.at[srcv.at[q, i]], rows.at[i % _NBUF],
                         gsem[i % _NBUF])

    def gather_wait(q, i):
        pltpu.make_async_copy(h_hbm.at[srcv.at[q, i]], rows.at[i % _NBUF],
                              gsem[i % _NBUF]).wait()

    def scat_start(q, i):
        pltpu.async_copy(rows.at[i % _NBUF], acc.at[dstv.at[q, i]],
                         ssem[i % _NBUF], add=True)

    def scat_wait(q, i):
        pltpu.make_async_copy(rows.at[i % _NBUF], acc.at[dstv.at[q, i]],
                              ssem[i % _NBUF]).wait()

    idx_start(0, 0)

    def macro(m, q):
        a = m + q

        @pl.when(a >= 1)
        def _():
            gather_wait(1 - q, _MAC - 1)
            scat_start(1 - q, _MAC - 1)

        @pl.when(a + 1 < nmac)
        def _():
            idx_start(1 - q, a + 1)

        idx_wait(q, a)
        for i in range(_MAC):
            for v in range(8):
                dv = dstraw[q, i, pl.ds(v * 16, 16)]
                loc = dv - c * _HALF
                inb = (loc >= 0) & (loc < _HALF)
                dmp = _HALF + lax.iota(jnp.int32, 16) + ((i + v) % 4) * 16
                dstv[q, i, pl.ds(v * 16, 16)] = jnp.where(inb, loc, dmp)
        for i in range(_MAC):
            if i >= 2:
                scat_wait(q, i - 2)
            else:
                @pl.when(a >= 1)
                def _():
                    scat_wait(1 - q, _MAC - 2 + i)
            gather_start(q, i)
            if i >= 1:
                gather_wait(q, i - 1)
                scat_start(q, i - 1)

        return 0

    def macro_pair(m, carry):
        macro(m, 0)
        macro(m, 1)
        return carry

    lax.fori_loop(0, nmac // 2, lambda t, cc: macro_pair(t * 2, cc), 0)
    qlast = (nmac - 1) % 2
    gather_wait(qlast, _MAC - 1)
    scat_start(qlast, _MAC - 1)
    scat_wait(qlast, _MAC - 2)
    scat_wait(qlast, _MAC - 1)
    plsc.subcore_barrier()

    @pl.when(s < 15)
    def _():
        pltpu.sync_copy(
            acc.at[pl.ds(s * _STRIPE, _STRIPE)],
            out_hbm.at[pl.ds(c * _HALF + s * _STRIPE, _STRIPE)],
        )

    @pl.when(s == 15)
    def _():
        pltpu.sync_copy(
            acc.at[pl.ds(15 * _STRIPE, _HALF - 15 * _STRIPE)],
            out_hbm.at[pl.ds(c * _HALF + 15 * _STRIPE, _HALF - 15 * _STRIPE)],
        )


_sc_params = pltpu.CompilerParams(use_tc_tiling_on_sc=False)


def _make_deg_call():
    return pl.kernel(
        _deg_body,
        out_type=jax.ShapeDtypeStruct((_N, 1), jnp.float32),
        mesh=_mesh,
        compiler_params=_sc_params,
        scratch_types=[
            pltpu.VMEM((_GRP, _K), jnp.int32),
            pltpu.VMEM((_GRP, _K), jnp.int32),
            pltpu.VMEM((_K, 1), jnp.float32),
            pltpu.VMEM_SHARED((_RD, 1), jnp.float32),
        ],
    )


def _make_agg_call():
    return pl.kernel(
        _agg_body,
        out_type=jax.ShapeDtypeStruct((_N, _D), jnp.float32),
        mesh=_mesh,
        compiler_params=_sc_params,
        scratch_types=[
            pltpu.VMEM((2, _MAC, _K), jnp.int32),
            pltpu.VMEM((2, _MAC, _K), jnp.int32),
            pltpu.VMEM((2, _MAC, _K), jnp.int32),
            pltpu.VMEM((_NBUF, _K, _D), jnp.float32),
            pltpu.VMEM_SHARED((_RD, _D), jnp.float32),
        ] + [pltpu.SemaphoreType.DMA] * 8,
    )


def _prologue_body(x_ref, w_ref, deg_ref, h_ref, dinv_ref):
    di = lax.rsqrt(deg_ref[...])
    dinv_ref[...] = di
    h = jnp.dot(x_ref[...], w_ref[...], preferred_element_type=jnp.float32)
    h_ref[...] = h * di


def _stats_body(a_ref, dinv_ref, b_ref, st_ref):
    y = jnp.maximum(a_ref[...] * dinv_ref[...] + b_ref[...], 0.0)
    s1 = jnp.sum(y, axis=0, keepdims=True)
    s2 = jnp.sum(y * y, axis=0, keepdims=True)
    blk = jnp.concatenate(
        [s1, s2, jnp.zeros((6, _D), jnp.float32)], axis=0
    )

    @pl.when(pl.program_id(0) == 0)
    def _():
        st_ref[...] = blk

    @pl.when(pl.program_id(0) > 0)
    def _():
        st_ref[...] = st_ref[...] + blk


def _apply_body(a_ref, dinv_ref, b_ref, st_ref, g_ref, be_ref, w_ref, o_ref):
    di = dinv_ref[...]
    y = jnp.maximum(a_ref[...] * di + b_ref[...], 0.0)
    mean = st_ref[0:1, :] * (1.0 / _N)
    var = st_ref[1:2, :] * (1.0 / _N) - mean * mean
    aa = g_ref[...] * lax.rsqrt(var + 1e-5)
    cc = be_ref[...] - mean * aa
    z = y * aa + cc
    o_ref[...] = jnp.dot(z, w_ref[...], preferred_element_type=jnp.float32) * di


def _pool_body(a_ref, dinv_ref, b_ref, st_ref, g_ref, be_ref, bt_ref, o_ref):
    y = jnp.maximum(a_ref[...] * dinv_ref[...] + b_ref[...], 0.0)
    mean = st_ref[0:1, :] * (1.0 / _N)
    var = st_ref[1:2, :] * (1.0 / _N) - mean * mean
    aa = g_ref[...] * lax.rsqrt(var + 1e-5)
    cc = be_ref[...] - mean * aa
    z = y * aa + cc
    onehot = (bt_ref[...] == lax.broadcasted_iota(jnp.int32, (1, _G), 1))
    part = lax.dot_general(
        onehot.astype(jnp.float32), z, (((0,), (0,)), ((), ())),
        preferred_element_type=jnp.float32,
    )

    @pl.when(pl.program_id(0) == 0)
    def _():
        o_ref[...] = part

    @pl.when(pl.program_id(0) > 0)
    def _():
        o_ref[...] = o_ref[...] + part


def _row_spec(width):
    return pl.BlockSpec((_BM, width), lambda i: (i, 0))


def _full_spec(shape):
    return pl.BlockSpec(shape, lambda i: tuple(0 for _ in shape))


def _prologue_call(x, w1, deg):
    nf = x.shape[1]
    return pl.pallas_call(
        _prologue_body,
        grid=(_TGRID,),
        in_specs=[_row_spec(nf), _full_spec((nf, _D)), _row_spec(1)],
        out_specs=[_row_spec(_D), _row_spec(1)],
        out_shape=[
            jax.ShapeDtypeStruct((_N, _D), jnp.float32),
            jax.ShapeDtypeStruct((_N, 1), jnp.float32),
        ],
    )(x, w1, deg)


def _stats_call(acc, dinv, b):
    return pl.pallas_call(
        _stats_body,
        grid=(_TGRID,),
        in_specs=[_row_spec(_D), _row_spec(1), _full_spec((1, _D))],
        out_specs=_full_spec((8, _D)),
        out_shape=jax.ShapeDtypeStruct((8, _D), jnp.float32),
    )(acc, dinv, b)


def _apply_call(acc, dinv, b, st, g, be, w):
    return pl.pallas_call(
        _apply_body,
        grid=(_TGRID,),
        in_specs=[
            _row_spec(_D), _row_spec(1), _full_spec((1, _D)),
            _full_spec((8, _D)), _full_spec((1, _D)), _full_spec((1, _D)),
            _full_spec((_D, _D)),
        ],
        out_specs=_row_spec(_D),
        out_shape=jax.ShapeDtypeStruct((_N, _D), jnp.float32),
    )(acc, dinv, b, st, g, be, w)


def _pool_call(acc, dinv, b, st, g, be, batch2d):
    return pl.pallas_call(
        _pool_body,
        grid=(_TGRID,),
        in_specs=[
            _row_spec(_D), _row_spec(1), _full_spec((1, _D)),
            _full_spec((8, _D)), _full_spec((1, _D)), _full_spec((1, _D)),
            _row_spec(1),
        ],
        out_specs=_full_spec((_G, _D)),
        out_shape=jax.ShapeDtypeStruct((_G, _D), jnp.float32),
    )(acc, dinv, b, st, g, be, batch2d)


def kernel(x, edge_index, batch, W1, b1, g1, be1, W2, b2, g2, be2,
           W3, b3, g3, be3, W4, b4, g4, be4):
    loop = jnp.arange(_N, dtype=jnp.int32)
    src = jnp.concatenate([edge_index[0].astype(jnp.int32), loop])
    dst = jnp.concatenate([edge_index[1].astype(jnp.int32), loop])
    pad = _ET_PAD - _ET
    src2d = jnp.concatenate([src, jnp.zeros((pad,), jnp.int32)]).reshape(_ROWS, _K)
    dst2d = jnp.concatenate([dst, jnp.full((pad,), _N, jnp.int32)]).reshape(_ROWS, _K)

    ones_col = jnp.ones((_K, 1), jnp.float32)
    zcol = jnp.zeros((_STRIPE, 1), jnp.float32)
    zrows = jnp.zeros((_STRIPE, _D), jnp.float32)
    batch2d = batch.astype(jnp.int32).reshape(_N, 1)

    deg = _make_deg_call()(dst2d, ones_col, zcol)
    h, dinv = _prologue_call(x, W1, deg)

    agg = _make_agg_call()
    params = [(b1, g1, be1, W2), (b2, g2, be2, W3), (b3, g3, be3, W4)]
    for b, g, be, w_next in params:
        a = agg(h, src2d, dst2d, zrows)
        st = _stats_call(a, dinv, b.reshape(1, _D))
        h = _apply_call(a, dinv, b.reshape(1, _D), st, g.reshape(1, _D),
                        be.reshape(1, _D), w_next)
    a = agg(h, src2d, dst2d, zrows)
    st = _stats_call(a, dinv, b4.reshape(1, _D))
    return _pool_call(a, dinv, b4.reshape(1, _D), st, g4.reshape(1, _D),
                      be4.reshape(1, _D), batch2d)


# speedup vs baseline: 1.4169x; 1.0039x over previous
import jax
import jax.numpy as jnp
from jax import lax
from jax.experimental import pallas as pl
from jax.experimental.pallas import tpu as pltpu
from jax.experimental.pallas import tpu_sc as plsc

_N = 50000
_E = 800000
_G = 128
_D = 64
_HALF = _N // 2
_NDUMP = 88
_RD = _HALF + _NDUMP
_STRIPE = _RD // 16
_K = 128
_GRP = 2
_MAC = 8
_NBUF = 2
_ET = _E + _N
_ROWS = 6656
_ET_PAD = _ROWS * _K
_RPT = _ROWS // 16
_OUTER = _RPT // _GRP

_BM = 2000
_TGRID = _N // _BM

_mesh = plsc.VectorSubcoreMesh(
    core_axis_name="c", subcore_axis_name="s", num_cores=2, num_subcores=16
)


def _deg_body(dst_hbm, ones_hbm, zcol_hbm, deg_hbm, dstbuf, idxbuf, onesv, dacc):
    c = lax.axis_index("c")
    s = lax.axis_index("s")
    pltpu.sync_copy(zcol_hbm, dacc.at[pl.ds(s * _STRIPE, _STRIPE)])
    pltpu.sync_copy(ones_hbm, onesv)
    plsc.subcore_barrier()
    row_base = s * _RPT

    def chunk(j, carry):
        r0 = row_base + j * _GRP
        pltpu.sync_copy(dst_hbm.at[pl.ds(r0, _GRP)], dstbuf)
        for g in range(_GRP):
            for i in range(8):
                dv = dstbuf[g, pl.ds(i * 16, 16)]
                loc = dv - c * _HALF
                inb = (loc >= 0) & (loc < _HALF)
                dmp = _HALF + lax.iota(jnp.int32, 16) + ((g + i) % 4) * 16
                idxbuf[g, pl.ds(i * 16, 16)] = jnp.where(inb, loc, dmp)
        for g in range(_GRP):
            pltpu.sync_copy(onesv, dacc.at[idxbuf.at[g]], add=True)
        return carry

    lax.fori_loop(0, _OUTER, chunk, 0)
    plsc.subcore_barrier()

    @pl.when(s < 15)
    def _():
        pltpu.sync_copy(
            dacc.at[pl.ds(s * _STRIPE, _STRIPE)],
            deg_hbm.at[pl.ds(c * _HALF + s * _STRIPE, _STRIPE)],
        )

    @pl.when(s == 15)
    def _():
        pltpu.sync_copy(
            dacc.at[pl.ds(15 * _STRIPE, _HALF - 15 * _STRIPE)],
            deg_hbm.at[pl.ds(c * _HALF + 15 * _STRIPE, _HALF - 15 * _STRIPE)],
        )


def _agg_body(h_hbm, src_hbm, dst_hbm, zrows_hbm, out_hbm,
              srcv, dstraw, dstv, rows, acc,
              isem0, isem1, gsem0, gsem1, gsem2, ssem0, ssem1, ssem2):
    c = lax.axis_index("c")
    s = lax.axis_index("s")
    isem = (isem0, isem1)
    gsem = (gsem0, gsem1, gsem2)
    ssem = (ssem0, ssem1, ssem2)
    pltpu.sync_copy(zrows_hbm, acc.at[pl.ds(s * _STRIPE, _STRIPE)])
    plsc.subcore_barrier()
    rb = s * _RPT
    nmac = _RPT // _MAC

    def idx_start(q, a):
        pltpu.async_copy(src_hbm.at[pl.ds(rb + a * _MAC, _MAC)],
                         srcv.at[q], isem[q])
        pltpu.async_copy(dst_hbm.at[pl.ds(rb + a * _MAC, _MAC)],
                         dstraw.at[q], isem[q])

    def idx_wait(q, a):
        pltpu.make_async_copy(src_hbm.at[pl.ds(rb + a * _MAC, _MAC)],
                              srcv.at[q], isem[q]).wait()
        pltpu.make_async_copy(dst_hbm.at[pl.ds(rb + a * _MAC, _MAC)],
                              dstraw.at[q], isem[q]).wait()

    def gather_start(q, i):
        pltpu.async_copy(h_hbm.at[srcv.at[q, i]], rows.at[i % _NBUF],
                         gsem[i % _NBUF])

    def gather_wait(q, i):
        pltpu.make_async_copy(h_hbm.at[srcv.at[q, i]], rows.at[i % _NBUF],
                              gsem[i % _NBUF]).wait()

    def scat_start(q, i):
        pltpu.async_copy(rows.at[i % _NBUF], acc.at[dstv.at[q, i]],
                         ssem[i % _NBUF], add=True)

    def scat_wait(q, i):
        pltpu.make_async_copy(rows.at[i % _NBUF], acc.at[dstv.at[q, i]],
                              ssem[i % _NBUF]).wait()

    idx_start(0, 0)

    def macro(m, q):
        a = m + q

        @pl.when(a >= 1)
        def _():
            gather_wait(1 - q, _MAC - 1)
            scat_start(1 - q, _MAC - 1)

        @pl.when(a + 1 < nmac)
        def _():
            idx_start(1 - q, a + 1)

        idx_wait(q, a)
        for i in range(_MAC):
            for v in range(8):
                dv = dstraw[q, i, pl.ds(v * 16, 16)]
                loc = dv - c * _HALF
                inb = (loc >= 0) & (loc < _HALF)
                dmp = _HALF + lax.iota(jnp.int32, 16) + ((i + v) % 4) * 16
                dstv[q, i, pl.ds(v * 16, 16)] = jnp.where(inb, loc, dmp)
        for i in range(_MAC):
            if i >= 2:
                scat_wait(q, i - 2)
            else:
                @pl.when(a >= 1)
                def _():
                    scat_wait(1 - q, _MAC - 2 + i)
            gather_start(q, i)
            if i >= 1:
                gather_wait(q, i - 1)
                scat_start(q, i - 1)

        return 0

    def macro_pair(m, carry):
        macro(m, 0)
        macro(m, 1)
        return carry

    lax.fori_loop(0, nmac // 2, lambda t, cc: macro_pair(t * 2, cc), 0)
    qlast = (nmac - 1) % 2
    gather_wait(qlast, _MAC - 1)
    scat_start(qlast, _MAC - 1)
    scat_wait(qlast, _MAC - 2)
    scat_wait(qlast, _MAC - 1)
    plsc.subcore_barrier()

    @pl.when(s < 15)
    def _():
        pltpu.sync_copy(
            acc.at[pl.ds(s * _STRIPE, _STRIPE)],
            out_hbm.at[pl.ds(c * _HALF + s * _STRIPE, _STRIPE)],
        )

    @pl.when(s == 15)
    def _():
        pltpu.sync_copy(
            acc.at[pl.ds(15 * _STRIPE, _HALF - 15 * _STRIPE)],
            out_hbm.at[pl.ds(c * _HALF + 15 * _STRIPE, _HALF - 15 * _STRIPE)],
        )


_sc_params = pltpu.CompilerParams(use_tc_tiling_on_sc=False)


def _make_deg_call():
    return pl.kernel(
        _deg_body,
        out_type=jax.ShapeDtypeStruct((_N, 1), jnp.float32),
        mesh=_mesh,
        compiler_params=_sc_params,
        scratch_types=[
            pltpu.VMEM((_GRP, _K), jnp.int32),
            pltpu.VMEM((_GRP, _K), jnp.int32),
            pltpu.VMEM((_K, 1), jnp.float32),
            pltpu.VMEM_SHARED((_RD, 1), jnp.float32),
        ],
    )


def _make_agg_call():
    return pl.kernel(
        _agg_body,
        out_type=jax.ShapeDtypeStruct((_N, _D), jnp.float32),
        mesh=_mesh,
        compiler_params=_sc_params,
        scratch_types=[
            pltpu.VMEM((2, _MAC, _K), jnp.int32),
            pltpu.VMEM((2, _MAC, _K), jnp.int32),
            pltpu.VMEM((2, _MAC, _K), jnp.int32),
            pltpu.VMEM((_NBUF, _K, _D), jnp.float32),
            pltpu.VMEM_SHARED((_RD, _D), jnp.float32),
        ] + [pltpu.SemaphoreType.DMA] * 8,
    )


def _prologue_body(x_ref, w_ref, deg_ref, h_ref, dinv_ref):
    di = lax.rsqrt(deg_ref[...])
    dinv_ref[...] = di
    h = jnp.dot(x_ref[...], w_ref[...], preferred_element_type=jnp.float32)
    h_ref[...] = h * di


def _stats_body(a_ref, dinv_ref, b_ref, st_ref):
    y = jnp.maximum(a_ref[...] * dinv_ref[...] + b_ref[...], 0.0)
    s1 = jnp.sum(y, axis=0, keepdims=True)
    s2 = jnp.sum(y * y, axis=0, keepdims=True)
    blk = jnp.concatenate(
        [s1, s2, jnp.zeros((6, _D), jnp.float32)], axis=0
    )

    @pl.when(pl.program_id(0) == 0)
    def _():
        st_ref[...] = blk

    @pl.when(pl.program_id(0) > 0)
    def _():
        st_ref[...] = st_ref[...] + blk


def _apply_body(a_ref, dinv_ref, b_ref, st_ref, g_ref, be_ref, w_ref, o_ref):
    di = dinv_ref[...]
    y = jnp.maximum(a_ref[...] * di + b_ref[...], 0.0)
    mean = st_ref[0:1, :] * (1.0 / _N)
    var = st_ref[1:2, :] * (1.0 / _N) - mean * mean
    aa = g_ref[...] * lax.rsqrt(var + 1e-5)
    cc = be_ref[...] - mean * aa
    z = y * aa + cc
    o_ref[...] = jnp.dot(z, w_ref[...], preferred_element_type=jnp.float32) * di


def _pool_body(a_ref, dinv_ref, b_ref, st_ref, g_ref, be_ref, bt_ref, o_ref):
    y = jnp.maximum(a_ref[...] * dinv_ref[...] + b_ref[...], 0.0)
    mean = st_ref[0:1, :] * (1.0 / _N)
    var = st_ref[1:2, :] * (1.0 / _N) - mean * mean
    aa = g_ref[...] * lax.rsqrt(var + 1e-5)
    cc = be_ref[...] - mean * aa
    z = y * aa + cc
    onehot = (bt_ref[...] == lax.broadcasted_iota(jnp.int32, (1, _G), 1))
    part = lax.dot_general(
        onehot.astype(jnp.float32), z, (((0,), (0,)), ((), ())),
        preferred_element_type=jnp.float32,
    )

    @pl.when(pl.program_id(0) == 0)
    def _():
        o_ref[...] = part

    @pl.when(pl.program_id(0) > 0)
    def _():
        o_ref[...] = o_ref[...] + part


def _row_spec(width):
    return pl.BlockSpec((_BM, width), lambda i: (i, 0))


def _full_spec(shape):
    return pl.BlockSpec(shape, lambda i: tuple(0 for _ in shape))


def _prologue_call(x, w1, deg):
    nf = x.shape[1]
    return pl.pallas_call(
        _prologue_body,
        grid=(_TGRID,),
        in_specs=[_row_spec(nf), _full_spec((nf, _D)), _row_spec(1)],
        out_specs=[_row_spec(_D), _row_spec(1)],
        out_shape=[
            jax.ShapeDtypeStruct((_N, _D), jnp.float32),
            jax.ShapeDtypeStruct((_N, 1), jnp.float32),
        ],
    )(x, w1, deg)


def _stats_call(acc, dinv, b):
    return pl.pallas_call(
        _stats_body,
        grid=(_TGRID,),
        in_specs=[_row_spec(_D), _row_spec(1), _full_spec((1, _D))],
        out_specs=_full_spec((8, _D)),
        out_shape=jax.ShapeDtypeStruct((8, _D), jnp.float32),
    )(acc, dinv, b)


def _apply_call(acc, dinv, b, st, g, be, w):
    return pl.pallas_call(
        _apply_body,
        grid=(_TGRID,),
        in_specs=[
            _row_spec(_D), _row_spec(1), _full_spec((1, _D)),
            _full_spec((8, _D)), _full_spec((1, _D)), _full_spec((1, _D)),
            _full_spec((_D, _D)),
        ],
        out_specs=_row_spec(_D),
        out_shape=jax.ShapeDtypeStruct((_N, _D), jnp.float32),
    )(acc, dinv, b, st, g, be, w)


def _pool_call(acc, dinv, b, st, g, be, batch2d):
    return pl.pallas_call(
        _pool_body,
        grid=(_TGRID,),
        in_specs=[
            _row_spec(_D), _row_spec(1), _full_spec((1, _D)),
            _full_spec((8, _D)), _full_spec((1, _D)), _full_spec((1, _D)),
            _row_spec(1),
        ],
        out_specs=_full_spec((_G, _D)),
        out_shape=jax.ShapeDtypeStruct((_G, _D), jnp.float32),
    )(acc, dinv, b, st, g, be, batch2d)


def kernel(x, edge_index, batch, W1, b1, g1, be1, W2, b2, g2, be2,
           W3, b3, g3, be3, W4, b4, g4, be4):
    loop = jnp.arange(_N, dtype=jnp.int32)
    src = jnp.concatenate([edge_index[0].astype(jnp.int32), loop])
    dst = jnp.concatenate([edge_index[1].astype(jnp.int32), loop])
    pad = _ET_PAD - _ET
    src2d = jnp.concatenate([src, jnp.zeros((pad,), jnp.int32)]).reshape(_ROWS, _K)
    dst2d = jnp.concatenate([dst, jnp.full((pad,), _N, jnp.int32)]).reshape(_ROWS, _K)

    ones_col = jnp.ones((_K, 1), jnp.float32)
    zcol = jnp.zeros((_STRIPE, 1), jnp.float32)
    zrows = jnp.zeros((_STRIPE, _D), jnp.float32)
    batch2d = batch.astype(jnp.int32).reshape(_N, 1)

    deg = _make_deg_call()(dst2d, ones_col, zcol)
    h, dinv = _prologue_call(x, W1, deg)

    agg = _make_agg_call()
    params = [(b1, g1, be1, W2), (b2, g2, be2, W3), (b3, g3, be3, W4)]
    for b, g, be, w_next in params:
        a = agg(h, src2d, dst2d, zrows)
        st = _stats_call(a, dinv, b.reshape(1, _D))
        h = _apply_call(a, dinv, b.reshape(1, _D), st, g.reshape(1, _D),
                        be.reshape(1, _D), w_next)
    a = agg(h, src2d, dst2d, zrows)
    st = _stats_call(a, dinv, b4.reshape(1, _D))
    return _pool_call(a, dinv, b4.reshape(1, _D), st, g4.reshape(1, _D),
                      be4.reshape(1, _D), batch2d)


# speedup vs baseline: 1.4967x; 1.0563x over previous
import jax
import jax.numpy as jnp
from jax import lax
from jax.experimental import pallas as pl
from jax.experimental.pallas import tpu as pltpu
from jax.experimental.pallas import tpu_sc as plsc

_N = 50000
_E = 800000
_G = 128
_D = 64
_HALF = _N // 2
_NDUMP = 88
_RD = _HALF + _NDUMP
_STRIPE = _RD // 16
_K = 128
_GRP = 2
_MAC = 8
_NBUF = 2
_ET = _E + _N
_ROWS = 6656
_ET_PAD = _ROWS * _K
_RPT = _ROWS // 16
_OUTER = _RPT // _GRP

_BM = 2000
_TGRID = _N // _BM

_mesh = plsc.VectorSubcoreMesh(
    core_axis_name="c", subcore_axis_name="s", num_cores=2, num_subcores=16
)


def _deg_body(dst_hbm, ones_hbm, zcol_hbm, deg_hbm, dstraw, idxb, onesv, dacc,
              isem0, isem1, ssem0, ssem1):
    c = lax.axis_index("c")
    s = lax.axis_index("s")
    isem = (isem0, isem1)
    ssem = (ssem0, ssem1)
    pltpu.sync_copy(zcol_hbm, dacc.at[pl.ds(s * _STRIPE, _STRIPE)])
    pltpu.sync_copy(ones_hbm, onesv)
    plsc.subcore_barrier()
    rb = s * _RPT
    nmac = _RPT // _MAC

    def idx_start(q, a):
        pltpu.async_copy(dst_hbm.at[pl.ds(rb + a * _MAC, _MAC)],
                         dstraw.at[q], isem[q])

    def idx_wait(q, a):
        pltpu.make_async_copy(dst_hbm.at[pl.ds(rb + a * _MAC, _MAC)],
                              dstraw.at[q], isem[q]).wait()

    def scat_start(q, i):
        pltpu.async_copy(onesv, dacc.at[idxb.at[q, i]], ssem[i % 2],
                         add=True)

    def scat_wait(q, i):
        pltpu.make_async_copy(onesv, dacc.at[idxb.at[q, i]],
                              ssem[i % 2]).wait()

    idx_start(0, 0)

    def macro(m, q):
        a = m + q

        @pl.when(a + 1 < nmac)
        def _():
            idx_start(1 - q, a + 1)

        idx_wait(q, a)
        for i in range(_MAC):
            for v in range(8):
                dv = dstraw[q, i, pl.ds(v * 16, 16)]
                loc = dv - c * _HALF
                inb = (loc >= 0) & (loc < _HALF)
                dmp = _HALF + lax.iota(jnp.int32, 16) + ((i + v) % 4) * 16
                idxb[q, i, pl.ds(v * 16, 16)] = jnp.where(inb, loc, dmp)
        for i in range(_MAC):
            if i >= 2:
                scat_wait(q, i - 2)
            else:
                @pl.when(a >= 1)
                def _():
                    scat_wait(1 - q, _MAC - 2 + i)
            scat_start(q, i)
        return 0

    def macro_pair(m, carry):
        macro(m, 0)
        macro(m, 1)
        return carry

    lax.fori_loop(0, nmac // 2, lambda t, cc: macro_pair(t * 2, cc), 0)
    qlast = (nmac - 1) % 2
    scat_wait(qlast, _MAC - 2)
    scat_wait(qlast, _MAC - 1)
    plsc.subcore_barrier()

    @pl.when(s < 15)
    def _():
        pltpu.sync_copy(
            dacc.at[pl.ds(s * _STRIPE, _STRIPE)],
            deg_hbm.at[pl.ds(c * _HALF + s * _STRIPE, _STRIPE)],
        )

    @pl.when(s == 15)
    def _():
        pltpu.sync_copy(
            dacc.at[pl.ds(15 * _STRIPE, _HALF - 15 * _STRIPE)],
            deg_hbm.at[pl.ds(c * _HALF + 15 * _STRIPE, _HALF - 15 * _STRIPE)],
        )


def _agg_body(h_hbm, src_hbm, dst_hbm, zrows_hbm, out_hbm,
              srcv, dstraw, dstv, rows, acc,
              isem0, isem1, gsem0, gsem1, gsem2, ssem0, ssem1, ssem2):
    c = lax.axis_index("c")
    s = lax.axis_index("s")
    isem = (isem0, isem1)
    gsem = (gsem0, gsem1, gsem2)
    ssem = (ssem0, ssem1, ssem2)
    pltpu.sync_copy(zrows_hbm, acc.at[pl.ds(s * _STRIPE, _STRIPE)])
    plsc.subcore_barrier()
    rb = s * _RPT
    nmac = _RPT // _MAC

    def idx_start(q, a):
        pltpu.async_copy(src_hbm.at[pl.ds(rb + a * _MAC, _MAC)],
                         srcv.at[q], isem[q])
        pltpu.async_copy(dst_hbm.at[pl.ds(rb + a * _MAC, _MAC)],
                         dstraw.at[q], isem[q])

    def idx_wait(q, a):
        pltpu.make_async_copy(src_hbm.at[pl.ds(rb + a * _MAC, _MAC)],
                              srcv.at[q], isem[q]).wait()
        pltpu.make_async_copy(dst_hbm.at[pl.ds(rb + a * _MAC, _MAC)],
                              dstraw.at[q], isem[q]).wait()

    def gather_start(q, i):
        pltpu.async_copy(h_hbm.at[srcv.at[q, i]], rows.at[i % _NBUF],
                         gsem[i % _NBUF])

    def gather_wait(q, i):
        pltpu.make_async_copy(h_hbm.at[srcv.at[q, i]], rows.at[i % _NBUF],
                              gsem[i % _NBUF]).wait()

    def scat_start(q, i):
        pltpu.async_copy(rows.at[i % _NBUF], acc.at[dstv.at[q, i]],
                         ssem[i % _NBUF], add=True)

    def scat_wait(q, i):
        pltpu.make_async_copy(rows.at[i % _NBUF], acc.at[dstv.at[q, i]],
                              ssem[i % _NBUF]).wait()

    idx_start(0, 0)

    def macro(m, q):
        a = m + q

        @pl.when(a >= 1)
        def _():
            gather_wait(1 - q, _MAC - 1)
            scat_start(1 - q, _MAC - 1)

        @pl.when(a + 1 < nmac)
        def _():
            idx_start(1 - q, a + 1)

        idx_wait(q, a)
        for i in range(_MAC):
            for v in range(8):
                dv = dstraw[q, i, pl.ds(v * 16, 16)]
                loc = dv - c * _HALF
                inb = (loc >= 0) & (loc < _HALF)
                dmp = _HALF + lax.iota(jnp.int32, 16) + ((i + v) % 4) * 16
                dstv[q, i, pl.ds(v * 16, 16)] = jnp.where(inb, loc, dmp)
        for i in range(_MAC):
            if i >= 2:
                scat_wait(q, i - 2)
            else:
                @pl.when(a >= 1)
                def _():
                    scat_wait(1 - q, _MAC - 2 + i)
            gather_start(q, i)
            if i >= 1:
                gather_wait(q, i - 1)
                scat_start(q, i - 1)

        return 0

    def macro_pair(m, carry):
        macro(m, 0)
        macro(m, 1)
        return carry

    lax.fori_loop(0, nmac // 2, lambda t, cc: macro_pair(t * 2, cc), 0)
    qlast = (nmac - 1) % 2
    gather_wait(qlast, _MAC - 1)
    scat_start(qlast, _MAC - 1)
    scat_wait(qlast, _MAC - 2)
    scat_wait(qlast, _MAC - 1)
    plsc.subcore_barrier()

    @pl.when(s < 15)
    def _():
        pltpu.sync_copy(
            acc.at[pl.ds(s * _STRIPE, _STRIPE)],
            out_hbm.at[pl.ds(c * _HALF + s * _STRIPE, _STRIPE)],
        )

    @pl.when(s == 15)
    def _():
        pltpu.sync_copy(
            acc.at[pl.ds(15 * _STRIPE, _HALF - 15 * _STRIPE)],
            out_hbm.at[pl.ds(c * _HALF + 15 * _STRIPE, _HALF - 15 * _STRIPE)],
        )


_sc_params = pltpu.CompilerParams(use_tc_tiling_on_sc=False)


def _make_deg_call():
    return pl.kernel(
        _deg_body,
        out_type=jax.ShapeDtypeStruct((_N, 1), jnp.float32),
        mesh=_mesh,
        compiler_params=_sc_params,
        scratch_types=[
            pltpu.VMEM((2, _MAC, _K), jnp.int32),
            pltpu.VMEM((2, _MAC, _K), jnp.int32),
            pltpu.VMEM((_K, 1), jnp.float32),
            pltpu.VMEM_SHARED((_RD, 1), jnp.float32),
        ] + [pltpu.SemaphoreType.DMA] * 4,
    )


def _make_agg_call():
    return pl.kernel(
        _agg_body,
        out_type=jax.ShapeDtypeStruct((_N, _D), jnp.float32),
        mesh=_mesh,
        compiler_params=_sc_params,
        scratch_types=[
            pltpu.VMEM((2, _MAC, _K), jnp.int32),
            pltpu.VMEM((2, _MAC, _K), jnp.int32),
            pltpu.VMEM((2, _MAC, _K), jnp.int32),
            pltpu.VMEM((_NBUF, _K, _D), jnp.float32),
            pltpu.VMEM_SHARED((_RD, _D), jnp.float32),
        ] + [pltpu.SemaphoreType.DMA] * 8,
    )


def _prologue_body(x_ref, w_ref, deg_ref, h_ref, dinv_ref):
    di = lax.rsqrt(deg_ref[...])
    dinv_ref[...] = di
    h = jnp.dot(x_ref[...], w_ref[...], preferred_element_type=jnp.float32)
    h_ref[...] = h * di


def _stats_body(a_ref, dinv_ref, b_ref, st_ref):
    y = jnp.maximum(a_ref[...] * dinv_ref[...] + b_ref[...], 0.0)
    s1 = jnp.sum(y, axis=0, keepdims=True)
    s2 = jnp.sum(y * y, axis=0, keepdims=True)
    blk = jnp.concatenate(
        [s1, s2, jnp.zeros((6, _D), jnp.float32)], axis=0
    )

    @pl.when(pl.program_id(0) == 0)
    def _():
        st_ref[...] = blk

    @pl.when(pl.program_id(0) > 0)
    def _():
        st_ref[...] = st_ref[...] + blk


def _apply_body(a_ref, dinv_ref, b_ref, st_ref, g_ref, be_ref, w_ref, o_ref):
    di = dinv_ref[...]
    y = jnp.maximum(a_ref[...] * di + b_ref[...], 0.0)
    mean = st_ref[0:1, :] * (1.0 / _N)
    var = st_ref[1:2, :] * (1.0 / _N) - mean * mean
    aa = g_ref[...] * lax.rsqrt(var + 1e-5)
    cc = be_ref[...] - mean * aa
    z = y * aa + cc
    o_ref[...] = jnp.dot(z, w_ref[...], preferred_element_type=jnp.float32) * di


def _pool_body(a_ref, dinv_ref, b_ref, st_ref, g_ref, be_ref, bt_ref, o_ref):
    y = jnp.maximum(a_ref[...] * dinv_ref[...] + b_ref[...], 0.0)
    mean = st_ref[0:1, :] * (1.0 / _N)
    var = st_ref[1:2, :] * (1.0 / _N) - mean * mean
    aa = g_ref[...] * lax.rsqrt(var + 1e-5)
    cc = be_ref[...] - mean * aa
    z = y * aa + cc
    onehot = (bt_ref[...] == lax.broadcasted_iota(jnp.int32, (1, _G), 1))
    part = lax.dot_general(
        onehot.astype(jnp.float32), z, (((0,), (0,)), ((), ())),
        preferred_element_type=jnp.float32,
    )

    @pl.when(pl.program_id(0) == 0)
    def _():
        o_ref[...] = part

    @pl.when(pl.program_id(0) > 0)
    def _():
        o_ref[...] = o_ref[...] + part


def _row_spec(width):
    return pl.BlockSpec((_BM, width), lambda i: (i, 0))


def _full_spec(shape):
    return pl.BlockSpec(shape, lambda i: tuple(0 for _ in shape))


def _prologue_call(x, w1, deg):
    nf = x.shape[1]
    return pl.pallas_call(
        _prologue_body,
        grid=(_TGRID,),
        in_specs=[_row_spec(nf), _full_spec((nf, _D)), _row_spec(1)],
        out_specs=[_row_spec(_D), _row_spec(1)],
        out_shape=[
            jax.ShapeDtypeStruct((_N, _D), jnp.float32),
            jax.ShapeDtypeStruct((_N, 1), jnp.float32),
        ],
    )(x, w1, deg)


def _stats_call(acc, dinv, b):
    return pl.pallas_call(
        _stats_body,
        grid=(_TGRID,),
        in_specs=[_row_spec(_D), _row_spec(1), _full_spec((1, _D))],
        out_specs=_full_spec((8, _D)),
        out_shape=jax.ShapeDtypeStruct((8, _D), jnp.float32),
    )(acc, dinv, b)


def _apply_call(acc, dinv, b, st, g, be, w):
    return pl.pallas_call(
        _apply_body,
        grid=(_TGRID,),
        in_specs=[
            _row_spec(_D), _row_spec(1), _full_spec((1, _D)),
            _full_spec((8, _D)), _full_spec((1, _D)), _full_spec((1, _D)),
            _full_spec((_D, _D)),
        ],
        out_specs=_row_spec(_D),
        out_shape=jax.ShapeDtypeStruct((_N, _D), jnp.float32),
    )(acc, dinv, b, st, g, be, w)


def _pool_call(acc, dinv, b, st, g, be, batch2d):
    return pl.pallas_call(
        _pool_body,
        grid=(_TGRID,),
        in_specs=[
            _row_spec(_D), _row_spec(1), _full_spec((1, _D)),
            _full_spec((8, _D)), _full_spec((1, _D)), _full_spec((1, _D)),
            _row_spec(1),
        ],
        out_specs=_full_spec((_G, _D)),
        out_shape=jax.ShapeDtypeStruct((_G, _D), jnp.float32),
    )(acc, dinv, b, st, g, be, batch2d)


def kernel(x, edge_index, batch, W1, b1, g1, be1, W2, b2, g2, be2,
           W3, b3, g3, be3, W4, b4, g4, be4):
    loop = jnp.arange(_N, dtype=jnp.int32)
    src = jnp.concatenate([edge_index[0].astype(jnp.int32), loop])
    dst = jnp.concatenate([edge_index[1].astype(jnp.int32), loop])
    pad = _ET_PAD - _ET
    src2d = jnp.concatenate([src, jnp.zeros((pad,), jnp.int32)]).reshape(_ROWS, _K)
    dst2d = jnp.concatenate([dst, jnp.full((pad,), _N, jnp.int32)]).reshape(_ROWS, _K)

    ones_col = jnp.ones((_K, 1), jnp.float32)
    zcol = jnp.zeros((_STRIPE, 1), jnp.float32)
    zrows = jnp.zeros((_STRIPE, _D), jnp.float32)
    batch2d = batch.astype(jnp.int32).reshape(_N, 1)

    deg = _make_deg_call()(dst2d, ones_col, zcol)
    h, dinv = _prologue_call(x, W1, deg)

    agg = _make_agg_call()
    params = [(b1, g1, be1, W2), (b2, g2, be2, W3), (b3, g3, be3, W4)]
    for b, g, be, w_next in params:
        a = agg(h, src2d, dst2d, zrows)
        st = _stats_call(a, dinv, b.reshape(1, _D))
        h = _apply_call(a, dinv, b.reshape(1, _D), st, g.reshape(1, _D),
                        be.reshape(1, _D), w_next)
    a = agg(h, src2d, dst2d, zrows)
    st = _stats_call(a, dinv, b4.reshape(1, _D))
    return _pool_call(a, dinv, b4.reshape(1, _D), st, g4.reshape(1, _D),
                      be4.reshape(1, _D), batch2d)
